# Initial kernel scaffold; baseline (speedup 1.0000x reference)
import jax
import jax.numpy as jnp
from jax.experimental import pallas as pl


def kernel(x, batch, W, b):
    raise NotImplementedError("write your pallas kernel here")


# speedup vs baseline: 1.8794x; 1.8794x over previous
import jax
import jax.numpy as jnp
from jax import lax
from jax.experimental import pallas as pl
from jax.experimental.pallas import tpu as pltpu
from jax.experimental.pallas import tpu_sc as plsc

N_NODES = 50000
HIDDEN = 256
SEGS = 512
OUT_DIM = 128

CHUNK = 80
NCHUNKS = N_NODES // CHUNK
NRR = 16
NC = 2
HH = HIDDEN // NC
CNT_W = 16
ACC = SEGS * HH
CNT = SEGS * CNT_W


def _sc_segment_sum(x_hbm, batch_hbm, sums_out, cnts_out,
                    accf, cntf, xbuf, idx_v):
    cid = lax.axis_index("c")
    sid = lax.axis_index("s")
    rr = sid
    colbase = cid * HH

    def zacc(i, carry):
        accf[pl.ds(i * 16, 16)] = jnp.zeros((16,), jnp.float32)
        return carry
    lax.fori_loop(0, ACC // 16, zacc, 0)

    def zcnt(i, carry):
        cntf[pl.ds(i * 16, 16)] = jnp.zeros((16,), jnp.float32)
        return carry
    lax.fori_loop(0, CNT // 16, zcnt, 0)

    start = rr * NCHUNKS // NRR
    end = (rr + 1) * NCHUNKS // NRR

    one16 = jnp.ones((16,), jnp.float32)

    def chunk(ci, carry):
        base = ci * CHUNK
        pltpu.sync_copy(batch_hbm.at[pl.ds(base, CHUNK)], idx_v)
        pltpu.sync_copy(x_hbm.at[pl.ds(base, CHUNK), pl.ds(colbase, HH)], xbuf)

        def group(g, carry2):
            idx_grp = idx_v[pl.ds(g * 16, 16)]
            for lane in range(16):
                s = idx_grp[lane]
                r = g * 16 + lane
                sb = s * HH
                for cg in range(HH // 16):
                    o = sb + cg * 16
                    accf[pl.ds(o, 16)] = (accf[pl.ds(o, 16)]
                                          + xbuf[r, pl.ds(cg * 16, 16)])
                cb = s * CNT_W
                cntf[pl.ds(cb, 16)] = cntf[pl.ds(cb, 16)] + one16
            return carry2
        lax.fori_loop(0, CHUNK // 16, group, 0)
        return carry
    lax.fori_loop(start, end, chunk, 0)

    pltpu.sync_copy(accf, sums_out.at[rr, cid])
    pltpu.sync_copy(cntf, cnts_out.at[rr, cid])


def _finish_kernel(sums_ref, cnts_ref, w_ref, b_ref, out_ref):
    s = jnp.sum(sums_ref[...], axis=0)
    c = jnp.sum(cnts_ref[...], axis=0)[0][:, 0:1]
    inv = 1.0 / jnp.clip(c, 1.0, None)
    m0 = s[0] * inv
    m1 = s[1] * inv
    dn = (((1,), (1,)), ((), ()))
    out = (lax.dot_general(m0, w_ref[:, 0:HH], dimension_numbers=dn,
                           preferred_element_type=jnp.float32)
           + lax.dot_general(m1, w_ref[:, HH:HIDDEN], dimension_numbers=dn,
                             preferred_element_type=jnp.float32))
    out_ref[...] = out + b_ref[...]


def kernel(x, batch, W, b):
    batch_i32 = batch.astype(jnp.int32)

    mesh = plsc.VectorSubcoreMesh(core_axis_name="c", subcore_axis_name="s")
    sc_call = pl.kernel(
        _sc_segment_sum,
        mesh=mesh,
        out_type=[
            jax.ShapeDtypeStruct((NRR, NC, ACC), jnp.float32),
            jax.ShapeDtypeStruct((NRR, NC, CNT), jnp.float32),
        ],
        scratch_types=[
            pltpu.VMEM((ACC,), jnp.float32),
            pltpu.VMEM((CNT,), jnp.float32),
            pltpu.VMEM((CHUNK, HH), jnp.float32),
            pltpu.VMEM((CHUNK,), jnp.int32),
        ],
    )
    sums, cnts = sc_call(x, batch_i32)
    sums = sums.reshape(NRR, NC, SEGS, HH)
    cnts = cnts.reshape(NRR, NC, SEGS, CNT_W)

    out = pl.pallas_call(
        _finish_kernel,
        out_shape=jax.ShapeDtypeStruct((SEGS, OUT_DIM), jnp.float32),
    )(sums, cnts, W, b.reshape(1, OUT_DIM))
    return out


# speedup vs baseline: 4.5432x; 2.4174x over previous
import jax
import jax.numpy as jnp
from jax import lax
from jax.experimental import pallas as pl
from jax.experimental.pallas import tpu as pltpu
from jax.experimental.pallas import tpu_sc as plsc

N_NODES = 50000
HIDDEN = 256
SEGS = 512
OUT_DIM = 128

CHUNK = 80
NCHUNKS = N_NODES // CHUNK
NRR = 16
NC = 2
HH = HIDDEN // NC
NCG = HH // 16
CNT_W = 16
ACC = SEGS * HH
CNT = SEGS * CNT_W


def _sc_segment_sum(x_hbm, batch_hbm, zsum_hbm, zcnt_hbm,
                    sums_out, cnts_out,
                    accf, cntf, xbuf, idx2, semx, semi):
    cid = lax.axis_index("c")
    sid = lax.axis_index("s")
    rr = sid
    colbase = cid * HH

    pltpu.sync_copy(zsum_hbm, accf)
    pltpu.sync_copy(zcnt_hbm, cntf)

    start = rr * NCHUNKS // NRR
    end = (rr + 1) * NCHUNKS // NRR

    def issue(ci, buf):
        base = ci * CHUNK
        pltpu.async_copy(batch_hbm.at[pl.ds(base, CHUNK)], idx2.at[buf], semi)
        pltpu.async_copy(x_hbm.at[pl.ds(base, CHUNK), pl.ds(colbase, HH)],
                         xbuf.at[buf], semx)

    def drain(buf):
        pltpu.make_async_copy(batch_hbm.at[pl.ds(0, CHUNK)],
                              idx2.at[buf], semi).wait()
        pltpu.make_async_copy(x_hbm.at[pl.ds(0, CHUNK), pl.ds(0, HH)],
                              xbuf.at[buf], semx).wait()

    issue(start, 0)

    one16 = jnp.ones((16,), jnp.float32)
    sixteen16 = jnp.full((16,), 16.0, jnp.float32)

    def chunk(ci, carry):
        k = ci - start
        par = lax.rem(k, 2)
        nxt = 1 - par
        drain(par)
        issue(jnp.minimum(ci + 1, end - 1), nxt)

        def group(g, carry2):
            idx_grp = idx2[par, pl.ds(g * 16, 16)]
            s0 = idx_grp[0]
            s15 = idx_grp[15]
            r0 = g * 16

            @pl.when(s0 == s15)
            def _fast():
                acc = [xbuf[par, r0, pl.ds(cg * 16, 16)] for cg in range(NCG)]
                for lane in range(1, 16):
                    for cg in range(NCG):
                        acc[cg] = acc[cg] + xbuf[par, r0 + lane,
                                                 pl.ds(cg * 16, 16)]
                sb = s0 * HH
                for cg in range(NCG):
                    o = sb + cg * 16
                    accf[pl.ds(o, 16)] = accf[pl.ds(o, 16)] + acc[cg]
                cb = s0 * CNT_W
                cntf[pl.ds(cb, 16)] = cntf[pl.ds(cb, 16)] + sixteen16

            @pl.when(s0 != s15)
            def _slow():
                for lane in range(16):
                    s = idx_grp[lane]
                    sb = s * HH
                    for cg in range(NCG):
                        o = sb + cg * 16
                        accf[pl.ds(o, 16)] = (accf[pl.ds(o, 16)]
                                              + xbuf[par, r0 + lane,
                                                     pl.ds(cg * 16, 16)])
                    cb = s * CNT_W
                    cntf[pl.ds(cb, 16)] = cntf[pl.ds(cb, 16)] + one16

            return carry2
        lax.fori_loop(0, CHUNK // 16, group, 0)
        return carry
    lax.fori_loop(start, end, chunk, 0)

    drain(lax.rem(end - start, 2))

    pltpu.sync_copy(accf, sums_out.at[rr, cid])
    pltpu.sync_copy(cntf, cnts_out.at[rr, cid])


def _finish_kernel(sums_ref, cnts_ref, w_ref, b_ref, out_ref):
    s = jnp.sum(sums_ref[...], axis=0)
    c = jnp.sum(cnts_ref[...], axis=0)[0][:, 0:1]
    inv = 1.0 / jnp.clip(c, 1.0, None)
    m0 = s[0] * inv
    m1 = s[1] * inv
    dn = (((1,), (1,)), ((), ()))
    out = (lax.dot_general(m0, w_ref[:, 0:HH], dimension_numbers=dn,
                           preferred_element_type=jnp.float32)
           + lax.dot_general(m1, w_ref[:, HH:HIDDEN], dimension_numbers=dn,
                             preferred_element_type=jnp.float32))
    out_ref[...] = out + b_ref[...]


def kernel(x, batch, W, b):
    batch_i32 = batch.astype(jnp.int32)
    zsum = jnp.zeros((ACC,), jnp.float32)
    zcnt = jnp.zeros((CNT,), jnp.float32)

    mesh = plsc.VectorSubcoreMesh(core_axis_name="c", subcore_axis_name="s")
    sc_call = pl.kernel(
        _sc_segment_sum,
        mesh=mesh,
        out_type=[
            jax.ShapeDtypeStruct((NRR, NC, ACC), jnp.float32),
            jax.ShapeDtypeStruct((NRR, NC, CNT), jnp.float32),
        ],
        scratch_types=[
            pltpu.VMEM((ACC,), jnp.float32),
            pltpu.VMEM((CNT,), jnp.float32),
            pltpu.VMEM((2, CHUNK, HH), jnp.float32),
            pltpu.VMEM((2, CHUNK), jnp.int32),
            pltpu.SemaphoreType.DMA,
            pltpu.SemaphoreType.DMA,
        ],
    )
    sums, cnts = sc_call(x, batch_i32, zsum, zcnt)
    sums = sums.reshape(NRR, NC, SEGS, HH)
    cnts = cnts.reshape(NRR, NC, SEGS, CNT_W)

    out = pl.pallas_call(
        _finish_kernel,
        out_shape=jax.ShapeDtypeStruct((SEGS, OUT_DIM), jnp.float32),
    )(sums, cnts, W, b.reshape(1, OUT_DIM))
    return out
